# baseline (speedup 1.0000x reference)
import functools

import jax
import jax.numpy as jnp
from jax import lax
from jax.experimental import pallas as pl
from jax.experimental.pallas import tpu as pltpu
from jax.experimental.pallas import tpu_sc as plsc

_NC = 2
_NS = 16
_NW = _NC * _NS
_C = 128
_NBUF = 2
_DD = 16


def _make_spmm(NP, D, E_pad, with_deg):
    PW = E_pad // _NW
    K = PW // _C
    NG = K // _NBUF
    RP = NP // _NS
    mesh = plsc.VectorSubcoreMesh(core_axis_name="c", subcore_axis_name="s")

    out_type = [jax.ShapeDtypeStruct((_NC, NP, D), jnp.float32)]
    scratch = [
        pltpu.VMEM((2, 2, _NBUF, _C), jnp.int32),
        pltpu.VMEM((_NBUF, _C, D), jnp.float32),
        pltpu.VMEM((8, D), jnp.float32),
        pltpu.VMEM_SHARED((NP, D), jnp.float32),
    ]
    nsem = 3 * _NBUF if with_deg else 2 * _NBUF
    if with_deg:
        scratch += [
            pltpu.VMEM((_C, _DD), jnp.float32),
            pltpu.VMEM((8, _DD), jnp.float32),
            pltpu.VMEM_SHARED((NP, _DD), jnp.float32),
        ]
        out_type.append(jax.ShapeDtypeStruct((_NC, NP, _DD), jnp.float32))
    scratch += [pltpu.SemaphoreType.DMA] * (nsem + 2)

    @functools.partial(
        pl.kernel,
        out_type=out_type,
        mesh=mesh,
        scratch_types=scratch,
        compiler_params=pltpu.CompilerParams(use_tc_tiling_on_sc=False),
    )
    def spmm(x_hbm, idx_hbm, *refs):
        if with_deg:
            (out_hbm, dout_hbm, idxb, gbuf, zbuf, acc, ones, zdbuf, dacc,
             *sems) = refs
            dsems = sems[2 * _NBUF:nsem]
        else:
            out_hbm, idxb, gbuf, zbuf, acc, *sems = refs
        gsems, ssems = sems[:_NBUF], sems[_NBUF:2 * _NBUF]
        isems = sems[nsem:]
        cid = lax.axis_index("c")
        sid = lax.axis_index("s")
        wid = sid * _NC + cid
        zv = jnp.zeros((16,), jnp.float32)
        for i in range(8):
            for j in range(D // 16):
                zbuf[i, pl.ds(j * 16, 16)] = zv
        for r in range(RP // 8):
            pltpu.sync_copy(zbuf, acc.at[pl.ds(sid * RP + r * 8, 8)])
        if with_deg:
            ov = jnp.ones((16,), jnp.float32)
            for i in range(_C):
                ones[i, :] = ov
            for i in range(8):
                zdbuf[i, :] = zv
            for r in range(RP // 8):
                pltpu.sync_copy(zdbuf, dacc.at[pl.ds(sid * RP + r * 8, 8)])
        plsc.subcore_barrier()

        def idx_fetch(g, u):
            pltpu.async_copy(idx_hbm.at[0, wid, g], idxb.at[u, 0], isems[u])
            pltpu.async_copy(idx_hbm.at[1, wid, g], idxb.at[u, 1], isems[u])

        def idx_wait(g, u):
            for cr in (0, 1):
                pltpu.make_async_copy(idx_hbm.at[cr, wid, g], idxb.at[u, cr],
                                      isems[u]).wait()

        def gather(u, b):
            pltpu.async_copy(x_hbm.at[idxb.at[u, 0, b]], gbuf.at[b], gsems[b])

        def gather_wait(u, b):
            pltpu.make_async_copy(x_hbm.at[idxb.at[u, 0, b]], gbuf.at[b],
                                  gsems[b]).wait()

        def scatter(u, b):
            pltpu.async_copy(gbuf.at[b], acc.at[idxb.at[u, 1, b]], ssems[b],
                             add=True)
            if with_deg:
                pltpu.async_copy(ones, dacc.at[idxb.at[u, 1, b]], dsems[b],
                                 add=True)

        def scatter_wait(u, b):
            pltpu.make_async_copy(gbuf.at[b], acc.at[idxb.at[u, 1, b]],
                                  ssems[b]).wait()
            if with_deg:
                pltpu.make_async_copy(ones, dacc.at[idxb.at[u, 1, b]],
                                      dsems[b]).wait()

        def run_group(u):
            for b in range(_NBUF):
                gather_wait(u, b)
                scatter(u, b)
            for b in range(_NBUF):
                scatter_wait(u, b)

        idx_fetch(0, 0)
        idx_wait(0, 0)
        idx_fetch(1, 1)
        for b in range(_NBUF):
            gather(0, b)

        def body(i, carry):
            run_group(0)
            idx_fetch(2 * i + 2, 0)
            idx_wait(2 * i + 1, 1)
            for b in range(_NBUF):
                gather(1, b)
            run_group(1)
            idx_fetch(2 * i + 3, 1)
            idx_wait(2 * i + 2, 0)
            for b in range(_NBUF):
                gather(0, b)
            return carry

        lax.fori_loop(0, NG // 2 - 1, body, 0)
        run_group(0)
        idx_wait(NG - 1, 1)
        for b in range(_NBUF):
            gather(1, b)
        run_group(1)

        plsc.subcore_barrier()
        pltpu.sync_copy(acc.at[pl.ds(sid * RP, RP)],
                        out_hbm.at[cid, pl.ds(sid * RP, RP)])
        if with_deg:
            pltpu.sync_copy(dacc.at[pl.ds(sid * RP, RP)],
                            dout_hbm.at[cid, pl.ds(sid * RP, RP)])

    return spmm


def _invd(dacc_blk):
    deg = dacc_blk[0, :, 0] + dacc_blk[1, :, 0]
    return 1.0 / jnp.maximum(deg, 1.0)


def _mid_body(s1_ref, dacc_ref, w1_ref, w2_ref, g_ref):
    s = s1_ref[0] + s1_ref[1]
    h = jnp.maximum(jnp.dot(s, w1_ref[...], preferred_element_type=jnp.float32), 0.0)
    u = jnp.dot(h, w2_ref[...], preferred_element_type=jnp.float32)
    g_ref[...] = u * _invd(dacc_ref[...])[:, None]


def _fin_body(s2_ref, dacc_ref, o_ref):
    o_ref[...] = (s2_ref[0] + s2_ref[1]) * _invd(dacc_ref[...])[:, None]


def kernel(x, edge_index, W1, W2):
    N, IN = x.shape
    H = W1.shape[1]
    E = edge_index.shape[1]
    NP = 10112
    Q = _NW * _C * _NBUF * 2
    E_pad = -(-E // Q) * Q
    PW = E_pad // _NW
    K = PW // _C
    NG = K // _NBUF

    P = E_pad - E
    ar = jnp.arange(P, dtype=jnp.int32)
    padblk = jnp.stack([ar % N, N + ar % (NP - N)])
    idxp = jnp.concatenate([jnp.flip(edge_index.astype(jnp.int32), 0), padblk],
                           axis=1).reshape(2, _NW, NG, _NBUF, _C)

    spmm1 = _make_spmm(NP, IN, E_pad, with_deg=True)
    spmm2 = _make_spmm(NP, IN, E_pad, with_deg=False)

    s1, dacc = spmm1(x, idxp)

    BN = 1264
    grid = (NP // BN,)
    g = pl.pallas_call(
        _mid_body,
        grid=grid,
        in_specs=[
            pl.BlockSpec((_NC, BN, IN), lambda i: (0, i, 0)),
            pl.BlockSpec((_NC, BN, _DD), lambda i: (0, i, 0)),
            pl.BlockSpec((IN, H), lambda i: (0, 0)),
            pl.BlockSpec((H, IN), lambda i: (0, 0)),
        ],
        out_specs=pl.BlockSpec((BN, IN), lambda i: (i, 0)),
        out_shape=jax.ShapeDtypeStruct((NP, IN), jnp.float32),
    )(s1, dacc, W1, W2)

    (s2,) = spmm2(g, idxp)

    o = pl.pallas_call(
        _fin_body,
        grid=grid,
        in_specs=[
            pl.BlockSpec((_NC, BN, IN), lambda i: (0, i, 0)),
            pl.BlockSpec((_NC, BN, _DD), lambda i: (0, i, 0)),
        ],
        out_specs=pl.BlockSpec((BN, IN), lambda i: (i, 0)),
        out_shape=jax.ShapeDtypeStruct((NP, IN), jnp.float32),
    )(s2, dacc)
    return o[:N]


# speedup vs baseline: 3.2990x; 3.2990x over previous
import functools

import jax
import jax.numpy as jnp
from jax import lax
from jax.experimental import pallas as pl
from jax.experimental.pallas import tpu as pltpu
from jax.experimental.pallas import tpu_sc as plsc

_NC = 2
_NS = 16
_NW = _NC * _NS
_C = 128
_NBUF = 2
_DD = 16


def _make_spmm(NP, D, E_pad, with_deg):
    PW = E_pad // _NW
    K = PW // _C
    NG = K // _NBUF
    RP = NP // _NS
    mesh = plsc.VectorSubcoreMesh(core_axis_name="c", subcore_axis_name="s")

    out_type = [jax.ShapeDtypeStruct((_NC, NP, D), jnp.float32)]
    scratch = [
        pltpu.VMEM((2, 2, _NBUF, _C), jnp.int32),
        pltpu.VMEM((_NBUF, _C, D), jnp.float32),
        pltpu.VMEM((8, D), jnp.float32),
        pltpu.VMEM_SHARED((NP, D), jnp.float32),
    ]
    nsem = 3 * _NBUF if with_deg else 2 * _NBUF
    if with_deg:
        scratch += [
            pltpu.VMEM((_C, _DD), jnp.float32),
            pltpu.VMEM((8, _DD), jnp.float32),
            pltpu.VMEM_SHARED((NP, _DD), jnp.float32),
        ]
        out_type.append(jax.ShapeDtypeStruct((_NC, NP, _DD), jnp.float32))
    scratch += [pltpu.SemaphoreType.DMA] * (nsem + 2)

    @functools.partial(
        pl.kernel,
        out_type=out_type,
        mesh=mesh,
        scratch_types=scratch,
        compiler_params=pltpu.CompilerParams(use_tc_tiling_on_sc=False),
    )
    def spmm(x_hbm, idx_hbm, *refs):
        if with_deg:
            (out_hbm, dout_hbm, idxb, gbuf, zbuf, acc, ones, zdbuf, dacc,
             *sems) = refs
            dsems = sems[2 * _NBUF:nsem]
        else:
            out_hbm, idxb, gbuf, zbuf, acc, *sems = refs
        gsems, ssems = sems[:_NBUF], sems[_NBUF:2 * _NBUF]
        isems = sems[nsem:]
        cid = lax.axis_index("c")
        sid = lax.axis_index("s")
        wid = sid * _NC + cid
        zv = jnp.zeros((16,), jnp.float32)
        for i in range(8):
            for j in range(D // 16):
                zbuf[i, pl.ds(j * 16, 16)] = zv
        for r in range(RP // 8):
            pltpu.sync_copy(zbuf, acc.at[pl.ds(sid * RP + r * 8, 8)])
        if with_deg:
            ov = jnp.ones((16,), jnp.float32)
            for i in range(_C):
                ones[i, :] = ov
            for i in range(8):
                zdbuf[i, :] = zv
            for r in range(RP // 8):
                pltpu.sync_copy(zdbuf, dacc.at[pl.ds(sid * RP + r * 8, 8)])
        plsc.subcore_barrier()

        def idx_fetch(g, u):
            pltpu.async_copy(idx_hbm.at[0, wid, g], idxb.at[u, 0], isems[u])
            pltpu.async_copy(idx_hbm.at[1, wid, g], idxb.at[u, 1], isems[u])

        def idx_wait(g, u):
            for cr in (0, 1):
                pltpu.make_async_copy(idx_hbm.at[cr, wid, g], idxb.at[u, cr],
                                      isems[u]).wait()

        def gather(u, b):
            pltpu.async_copy(x_hbm.at[idxb.at[u, 1, b]], gbuf.at[b], gsems[b])

        def gather_wait(u, b):
            pltpu.make_async_copy(x_hbm.at[idxb.at[u, 1, b]], gbuf.at[b],
                                  gsems[b]).wait()

        def scatter(u, b):
            pltpu.async_copy(gbuf.at[b], acc.at[idxb.at[u, 0, b]], ssems[b],
                             add=True)
            if with_deg:
                pltpu.async_copy(ones, dacc.at[idxb.at[u, 0, b]], dsems[b],
                                 add=True)

        def scatter_wait(u, b):
            pltpu.make_async_copy(gbuf.at[b], acc.at[idxb.at[u, 0, b]],
                                  ssems[b]).wait()
            if with_deg:
                pltpu.make_async_copy(ones, dacc.at[idxb.at[u, 0, b]],
                                      dsems[b]).wait()

        def run_group(u):
            for b in range(_NBUF):
                gather_wait(u, b)
                scatter(u, b)
            for b in range(_NBUF):
                scatter_wait(u, b)

        idx_fetch(0, 0)
        idx_wait(0, 0)
        idx_fetch(1, 1)
        for b in range(_NBUF):
            gather(0, b)

        def body(i, carry):
            run_group(0)
            idx_fetch(2 * i + 2, 0)
            idx_wait(2 * i + 1, 1)
            for b in range(_NBUF):
                gather(1, b)
            run_group(1)
            idx_fetch(2 * i + 3, 1)
            idx_wait(2 * i + 2, 0)
            for b in range(_NBUF):
                gather(0, b)
            return carry

        lax.fori_loop(0, NG // 2 - 1, body, 0)
        run_group(0)
        idx_wait(NG - 1, 1)
        for b in range(_NBUF):
            gather(1, b)
        run_group(1)

        plsc.subcore_barrier()
        pltpu.sync_copy(acc.at[pl.ds(sid * RP, RP)],
                        out_hbm.at[cid, pl.ds(sid * RP, RP)])
        if with_deg:
            pltpu.sync_copy(dacc.at[pl.ds(sid * RP, RP)],
                            dout_hbm.at[cid, pl.ds(sid * RP, RP)])

    return spmm


def _invd(dacc_blk):
    deg = dacc_blk[0, :, 0] + dacc_blk[1, :, 0]
    return 1.0 / jnp.maximum(deg, 1.0)


def _mid_body(s1_ref, dacc_ref, w1_ref, w2_ref, g_ref):
    s = s1_ref[0] + s1_ref[1]
    h = jnp.maximum(jnp.dot(s, w1_ref[...], preferred_element_type=jnp.float32), 0.0)
    u = jnp.dot(h, w2_ref[...], preferred_element_type=jnp.float32)
    g_ref[...] = u * _invd(dacc_ref[...])[:, None]


def _fin_body(s2_ref, dacc_ref, o_ref):
    o_ref[...] = (s2_ref[0] + s2_ref[1]) * _invd(dacc_ref[...])[:, None]


def kernel(x, edge_index, W1, W2):
    N, IN = x.shape
    H = W1.shape[1]
    E = edge_index.shape[1]
    NP = 10112
    Q = _NW * _C * _NBUF * 2
    E_pad = -(-E // Q) * Q
    PW = E_pad // _NW
    K = PW // _C
    NG = K // _NBUF

    P = E_pad - E
    ar = jnp.arange(P, dtype=jnp.int32)
    padblk = jnp.stack([N + ar % (NP - N), ar % N])
    idxp = jnp.concatenate([edge_index.astype(jnp.int32), padblk],
                           axis=1).reshape(2, _NW, NG, _NBUF, _C)

    spmm1 = _make_spmm(NP, IN, E_pad, with_deg=True)
    spmm2 = _make_spmm(NP, IN, E_pad, with_deg=False)

    s1, dacc = spmm1(x, idxp)

    BN = 1264
    grid = (NP // BN,)
    g = pl.pallas_call(
        _mid_body,
        grid=grid,
        in_specs=[
            pl.BlockSpec((_NC, BN, IN), lambda i: (0, i, 0)),
            pl.BlockSpec((_NC, BN, _DD), lambda i: (0, i, 0)),
            pl.BlockSpec((IN, H), lambda i: (0, 0)),
            pl.BlockSpec((H, IN), lambda i: (0, 0)),
        ],
        out_specs=pl.BlockSpec((BN, IN), lambda i: (i, 0)),
        out_shape=jax.ShapeDtypeStruct((NP, IN), jnp.float32),
    )(s1, dacc, W1, W2)

    (s2,) = spmm2(g, idxp)

    o = pl.pallas_call(
        _fin_body,
        grid=grid,
        in_specs=[
            pl.BlockSpec((_NC, BN, IN), lambda i: (0, i, 0)),
            pl.BlockSpec((_NC, BN, _DD), lambda i: (0, i, 0)),
        ],
        out_specs=pl.BlockSpec((BN, IN), lambda i: (i, 0)),
        out_shape=jax.ShapeDtypeStruct((NP, IN), jnp.float32),
    )(s2, dacc)
    return o[:N]
